# baseline (speedup 1.0000x reference)
import functools

import jax
import jax.numpy as jnp
from jax.experimental import pallas as pl
from jax.experimental.pallas import tpu as pltpu
from jax.experimental.pallas import tpu_sc as plsc

_N_SPARSE = 26
_VOCAB = 100000
_EMB = 16
_B = 16384
_DENSE = 13
_H1, _H2, _OUT = 256, 128, 64
_FPAD = 32
_NQ = 4
_BGRP = 16
_NWIN = _NQ * (_B // _BGRP)
_BM = 1024


def _sc_gather4(tab_flat, idx_win):
    mesh = plsc.VectorSubcoreMesh(core_axis_name="core", subcore_axis_name="subcore")

    @functools.partial(
        pl.kernel,
        out_type=jax.ShapeDtypeStruct((_NWIN, 128, _EMB), jnp.float32),
        mesh=mesh,
        compiler_params=pltpu.CompilerParams(use_tc_tiling_on_sc=False),
    )
    def k(tab_hbm, idx_hbm, o_hbm):
        def body(i_vmem, o_vmem):
            pltpu.sync_copy(tab_hbm.at[i_vmem.at[0, 0]], o_vmem.at[0])

        pltpu.emit_pipeline(
            body,
            grid=(_NWIN,),
            in_specs=[pl.BlockSpec((1, 1, 128), index_map=lambda w: (w, 0, 0))],
            out_specs=[pl.BlockSpec((1, 128, _EMB),
                                    index_map=lambda w: (w, 0, 0))],
            core_axis_name=("core", "subcore"),
            dimension_semantics=(pltpu.PARALLEL,),
        )(idx_hbm, o_hbm)

    return k(tab_flat, idx_win)


def _mlp_body(x4_ref, dense_ref, w1e_ref, w1b_ref, b1_ref, w2_ref, b2_ref,
              wo_ref, bo_ref, o_ref):
    dn = (((1,), (0,)), ((), ()))
    f32 = jnp.float32
    h = jax.lax.dot_general(x4_ref[0], w1e_ref[pl.ds(0, 128), :], dn,
                            preferred_element_type=f32)
    for q in range(1, _NQ):
        h = h + jax.lax.dot_general(x4_ref[q], w1e_ref[pl.ds(128 * q, 128), :],
                                    dn, preferred_element_type=f32)
    h = h + jax.lax.dot_general(dense_ref[...], w1b_ref[...], dn,
                                preferred_element_type=f32)
    h = jnp.maximum(h + b1_ref[...], 0.0)
    h = jax.lax.dot_general(h, w2_ref[...], dn, preferred_element_type=f32)
    h = jnp.maximum(h + b2_ref[...], 0.0)
    out = jax.lax.dot_general(h, wo_ref[...], dn, preferred_element_type=f32)
    out = out + bo_ref[...]
    ssq = jnp.sum(out * out, axis=1, keepdims=True)
    denom = jnp.maximum(jnp.sqrt(ssq), 1e-12)
    o_ref[...] = out / denom


def _tc_mlp(x4, dense_0, W1, b1, W2, b2, Wout, bout):
    w1a = W1[:_N_SPARSE * _EMB].reshape(_N_SPARSE, _EMB, _H1)
    w1e = jnp.pad(w1a, ((0, _FPAD - _N_SPARSE), (0, 0), (0, 0))).reshape(
        _FPAD * _EMB, _H1)
    w1b = W1[_N_SPARSE * _EMB:]
    full = lambda shape: pl.BlockSpec(shape, lambda i: tuple(0 for _ in shape))
    return pl.pallas_call(
        _mlp_body,
        grid=(_B // _BM,),
        in_specs=[
            pl.BlockSpec((_NQ, _BM, 128), lambda i: (0, i, 0)),
            pl.BlockSpec((_BM, _DENSE), lambda i: (i, 0)),
            full((_FPAD * _EMB, _H1)),
            full((_DENSE, _H1)),
            full((1, _H1)),
            full((_H1, _H2)),
            full((1, _H2)),
            full((_H2, _OUT)),
            full((1, _OUT)),
        ],
        out_specs=pl.BlockSpec((_BM, _OUT), lambda i: (i, 0)),
        out_shape=jax.ShapeDtypeStruct((_B, _OUT), jnp.float32),
    )(x4, dense_0, w1e, w1b, b1[None, :], W2, b2[None, :], Wout, bout[None, :])


def kernel(sparse_0, sparse_1, sparse_2, sparse_3, sparse_4, sparse_5,
           sparse_6, sparse_7, sparse_8, sparse_9, sparse_10, sparse_11,
           sparse_12, sparse_13, sparse_14, sparse_15, sparse_16, sparse_17,
           sparse_18, sparse_19, sparse_20, sparse_21, sparse_22, sparse_23,
           sparse_24, sparse_25, dense_0, tables, W1, b1, W2, b2, Wout, bout):
    sparse = [sparse_0, sparse_1, sparse_2, sparse_3, sparse_4, sparse_5,
              sparse_6, sparse_7, sparse_8, sparse_9, sparse_10, sparse_11,
              sparse_12, sparse_13, sparse_14, sparse_15, sparse_16,
              sparse_17, sparse_18, sparse_19, sparse_20, sparse_21,
              sparse_22, sparse_23, sparse_24, sparse_25]
    idx = jnp.stack(sparse, axis=1)
    offs = (jnp.arange(_N_SPARSE, dtype=jnp.int32) * _VOCAB)[None, :]
    idx = jnp.pad(idx + offs, ((0, 0), (0, _FPAD - _N_SPARSE)))
    idx_win = idx.reshape(_B // _BGRP, _BGRP, _NQ, 8).transpose(2, 0, 1, 3)
    idx_win = idx_win.reshape(_NWIN, 1, 128)
    tab_flat = tables.reshape(_N_SPARSE * _VOCAB, _EMB)
    x4 = _sc_gather4(tab_flat, idx_win).reshape(_NQ, _B, 8 * _EMB)
    return _tc_mlp(x4, dense_0, W1, b1, W2, b2, Wout, bout)
